# baseline (speedup 1.0000x reference)
import jax
import jax.numpy as jnp
from jax import lax
from jax.experimental import pallas as pl
from jax.experimental.pallas import tpu as pltpu
from jax.experimental.pallas import tpu_sc as plsc

N = 10000
E = 320000
D_IN = 128
H1 = 128
H2 = 64
NPG = 100
OUT_DIM = 10

NP = 10240
DUMMY = N
CHUNK = 128
NCHUNK = 2560
EP = NCHUNK * CHUNK
NC = 2
NS = 16
RPT = NP // NS
RB = 1280
GRID = NP // RB

_mesh = plsc.VectorSubcoreMesh(core_axis_name="c", subcore_axis_name="s")
_f32 = jnp.float32
_sc_params = pltpu.CompilerParams(use_tc_tiling_on_sc=False)


def _make_deg():
  cpt = NCHUNK // (NC * NS)

  def body(dst_hbm, ones_hbm, zeros_hbm, deg_out, ones_v, dsts_v, deg_sh):
    c = lax.axis_index("c")
    s = lax.axis_index("s")
    wid = s * NC + c
    pltpu.sync_copy(zeros_hbm.at[pl.ds(s * RPT, RPT)],
                    deg_sh.at[pl.ds(s * RPT, RPT)])
    pltpu.sync_copy(ones_hbm, ones_v)
    pltpu.sync_copy(dst_hbm.at[pl.ds(wid * cpt, cpt)], dsts_v)
    plsc.subcore_barrier()

    def chunk(j, carry):
      pltpu.sync_copy(ones_v, deg_sh.at[dsts_v.at[j]], add=True)
      return carry

    lax.fori_loop(0, cpt, chunk, 0)
    plsc.subcore_barrier()
    pltpu.sync_copy(deg_sh.at[pl.ds(s * RPT, RPT)],
                    deg_out.at[c, pl.ds(s * RPT, RPT)])

  return pl.kernel(
      body,
      out_type=jax.ShapeDtypeStruct((NC, NP), _f32),
      mesh=_mesh,
      scratch_types=[
          pltpu.VMEM((CHUNK,), _f32),
          pltpu.VMEM((cpt, CHUNK), jnp.int32),
          pltpu.VMEM_SHARED((NP,), _f32),
      ],
  )


_deg_call = _make_deg()


def _make_agg(width):
  cpt = NCHUNK // NS
  nidx = 16
  ngrp = cpt // nidx

  def body(hs_hbm, src_hbm, dst_hbm, out_hbm,
           srcs_v, dsts_v, rows0, rows1, semg0, semg1, sems0, sems1,
           tab_sh, acc_sh):
    c = lax.axis_index("c")
    s = lax.axis_index("s")
    r0 = s * RPT
    pltpu.sync_copy(hs_hbm.at[c, pl.ds(r0, RPT)], tab_sh.at[pl.ds(r0, RPT)])
    pltpu.sync_copy(hs_hbm.at[c, pl.ds(r0, RPT)], acc_sh.at[pl.ds(r0, RPT)])
    plsc.subcore_barrier()
    base = s * cpt

    def start_g(buf, idx_row, semg):
      return pltpu.async_copy(tab_sh.at[idx_row], buf, semg)

    def wait_g(buf, semg):
      pltpu.make_async_copy(tab_sh.at[pl.ds(0, CHUNK)], buf, semg).wait()

    def start_s(buf, idx_row, sems):
      return pltpu.async_copy(buf, acc_sh.at[idx_row], sems, add=True)

    def wait_s(buf, sems):
      pltpu.make_async_copy(buf, acc_sh.at[pl.ds(0, CHUNK)], sems).wait()

    def group(g, carry):
      gbase = base + g * nidx
      pltpu.sync_copy(src_hbm.at[pl.ds(gbase, nidx)], srcs_v)
      pltpu.sync_copy(dst_hbm.at[pl.ds(gbase, nidx)], dsts_v)
      start_g(rows0, srcs_v.at[0], semg0)
      start_g(rows1, srcs_v.at[1], semg1)

      def pair(k, c2):
        j0 = 2 * k + 2
        wait_g(rows0, semg0)
        start_s(rows0, dsts_v.at[j0 - 2], sems0)
        wait_g(rows1, semg1)
        start_s(rows1, dsts_v.at[j0 - 1], sems1)
        wait_s(rows0, sems0)
        start_g(rows0, srcs_v.at[j0], semg0)
        wait_s(rows1, sems1)
        start_g(rows1, srcs_v.at[j0 + 1], semg1)
        return c2

      lax.fori_loop(0, nidx // 2 - 1, pair, 0)
      wait_g(rows0, semg0)
      start_s(rows0, dsts_v.at[nidx - 2], sems0)
      wait_g(rows1, semg1)
      start_s(rows1, dsts_v.at[nidx - 1], sems1)
      wait_s(rows0, sems0)
      wait_s(rows1, sems1)
      return carry

    lax.fori_loop(0, ngrp, group, 0)
    plsc.subcore_barrier()
    pltpu.sync_copy(acc_sh.at[pl.ds(r0, RPT)], out_hbm.at[c, pl.ds(r0, RPT)])

  return pl.kernel(
      body,
      out_type=jax.ShapeDtypeStruct((NC, NP, width), _f32),
      mesh=_mesh,
      compiler_params=_sc_params,
      scratch_types=[
          pltpu.VMEM((nidx, CHUNK), jnp.int32),
          pltpu.VMEM((nidx, CHUNK), jnp.int32),
          pltpu.VMEM((CHUNK, width), _f32),
          pltpu.VMEM((CHUNK, width), _f32),
          pltpu.SemaphoreType.DMA,
          pltpu.SemaphoreType.DMA,
          pltpu.SemaphoreType.DMA,
          pltpu.SemaphoreType.DMA,
          pltpu.VMEM_SHARED((NP, width), _f32),
          pltpu.VMEM_SHARED((NP, width), _f32),
      ],
  )


_agg64 = _make_agg(H1 // NC)
_agg32 = _make_agg(H2 // NC)


def _k2_body(deg_ref, x_ref, w1_ref, hs_ref, dinv_ref):
  d = deg_ref[...]
  deg = d[0] + d[1] + 1.0
  dinv = lax.rsqrt(deg)[:, None]
  h = jnp.dot(x_ref[...], w1_ref[...], preferred_element_type=_f32)
  hs = h * dinv
  hs_ref[0] = hs[:, : H1 // 2]
  hs_ref[1] = hs[:, H1 // 2:]
  dinv_ref[...] = dinv


def _k2_call(degp, xp, W1):
  return pl.pallas_call(
      _k2_body,
      grid=(GRID,),
      in_specs=[
          pl.BlockSpec((NC, RB), lambda i: (0, i)),
          pl.BlockSpec((RB, D_IN), lambda i: (i, 0)),
          pl.BlockSpec((D_IN, H1), lambda i: (0, 0)),
      ],
      out_specs=[
          pl.BlockSpec((NC, RB, H1 // 2), lambda i: (0, i, 0)),
          pl.BlockSpec((RB, 1), lambda i: (i, 0)),
      ],
      out_shape=[
          jax.ShapeDtypeStruct((NC, NP, H1 // 2), _f32),
          jax.ShapeDtypeStruct((NP, 1), _f32),
      ],
  )(degp, xp, W1)


def _k4_body(agg_ref, dinv_ref, b1_ref, w2_ref, out_ref):
  a = agg_ref[...]
  agg = jnp.concatenate([a[0], a[1]], axis=1)
  dinv = dinv_ref[...]
  h1 = jnp.maximum(agg * dinv + b1_ref[...], 0.0)
  gs = jnp.dot(h1, w2_ref[...], preferred_element_type=_f32) * dinv
  out_ref[0] = gs[:, : H2 // 2]
  out_ref[1] = gs[:, H2 // 2:]


def _k4_call(agg1, dinv, b1, W2):
  return pl.pallas_call(
      _k4_body,
      grid=(GRID,),
      in_specs=[
          pl.BlockSpec((NC, RB, H1 // 2), lambda i: (0, i, 0)),
          pl.BlockSpec((RB, 1), lambda i: (i, 0)),
          pl.BlockSpec((1, H1), lambda i: (0, 0)),
          pl.BlockSpec((H1, H2), lambda i: (0, 0)),
      ],
      out_specs=pl.BlockSpec((NC, RB, H2 // 2), lambda i: (0, i, 0)),
      out_shape=jax.ShapeDtypeStruct((NC, NP, H2 // 2), _f32),
  )(agg1, dinv, b1, W2)


def _k6_body(agg_ref, dinv_ref, b2_ref, out_ref):
  a = agg_ref[...]
  agg = jnp.concatenate([a[0], a[1]], axis=1)
  h2 = agg * dinv_ref[...] + b2_ref[...]
  out_ref[...] = jnp.maximum(h2, 0.0)


def _k6_call(agg2, dinv, b2):
  return pl.pallas_call(
      _k6_body,
      grid=(GRID,),
      in_specs=[
          pl.BlockSpec((NC, RB, H2 // 2), lambda i: (0, i, 0)),
          pl.BlockSpec((RB, 1), lambda i: (i, 0)),
          pl.BlockSpec((1, H2), lambda i: (0, 0)),
      ],
      out_specs=pl.BlockSpec((RB, H2), lambda i: (i, 0)),
      out_shape=jax.ShapeDtypeStruct((NP, H2), _f32),
  )(agg2, dinv, b2)


def _head_body(z_ref, wl_ref, bl_ref, out_ref):
  logits = jnp.dot(z_ref[...], wl_ref[...], preferred_element_type=_f32)
  logits = logits + bl_ref[...]
  m = jnp.max(logits, axis=1, keepdims=True)
  lse = jnp.log(jnp.sum(jnp.exp(logits - m), axis=1, keepdims=True)) + m
  out_ref[...] = logits - lse


def _head_call(z, Wl, bl):
  return pl.pallas_call(
      _head_body,
      out_shape=jax.ShapeDtypeStruct((NPG, OUT_DIM), _f32),
  )(z, Wl, bl)


def kernel(x, edge_index, W1, b1, W2, b2, Wl, bl):
  xp = jnp.concatenate([x, jnp.zeros((NP - N, D_IN), _f32)], axis=0)
  pad = jnp.full((EP - E,), DUMMY, jnp.int32)
  src2 = jnp.concatenate([edge_index[0], pad]).reshape(NCHUNK, CHUNK)
  dst2 = jnp.concatenate([edge_index[1], pad]).reshape(NCHUNK, CHUNK)
  ones_rows = jnp.ones((CHUNK,), _f32)
  zeros_deg = jnp.zeros((NP,), _f32)

  degp = _deg_call(dst2, ones_rows, zeros_deg)
  hs, dinv = _k2_call(degp, xp, W1)
  agg1 = _agg64(hs, src2, dst2)
  gs = _k4_call(agg1, dinv, b1.reshape(1, H1), W2)
  agg2 = _agg32(gs, src2, dst2)
  h2 = _k6_call(agg2, dinv, b2.reshape(1, H2))
  z = h2[:N].reshape(NPG, H2 * NPG)
  return _head_call(z, Wl, bl.reshape(1, OUT_DIM))


# speedup vs baseline: 1.0872x; 1.0872x over previous
import jax
import jax.numpy as jnp
from jax import lax
from jax.experimental import pallas as pl
from jax.experimental.pallas import tpu as pltpu
from jax.experimental.pallas import tpu_sc as plsc

N = 10000
E = 320000
D_IN = 128
H1 = 128
H2 = 64
NPG = 100
OUT_DIM = 10

NP = 10240
DUMMY = N
CHUNK = 128
NCHUNK = 2560
EP = NCHUNK * CHUNK
NC = 2
NS = 16
RPT = NP // NS
RB = 1280
GRID = NP // RB

_mesh = plsc.VectorSubcoreMesh(core_axis_name="c", subcore_axis_name="s")
_f32 = jnp.float32
_sc_params = pltpu.CompilerParams(use_tc_tiling_on_sc=False)


def _make_deg():
  cpt = NCHUNK // (NC * NS)

  def body(dst_hbm, ones_hbm, zeros_hbm, deg_out, ones_v, dsts_v, deg_sh):
    c = lax.axis_index("c")
    s = lax.axis_index("s")
    wid = s * NC + c
    pltpu.sync_copy(zeros_hbm.at[pl.ds(s * RPT, RPT)],
                    deg_sh.at[pl.ds(s * RPT, RPT)])
    pltpu.sync_copy(ones_hbm, ones_v)
    pltpu.sync_copy(dst_hbm.at[pl.ds(wid * cpt, cpt)], dsts_v)
    plsc.subcore_barrier()

    def chunk(j, carry):
      pltpu.sync_copy(ones_v, deg_sh.at[dsts_v.at[j]], add=True)
      return carry

    lax.fori_loop(0, cpt, chunk, 0)
    plsc.subcore_barrier()
    pltpu.sync_copy(deg_sh.at[pl.ds(s * RPT, RPT)],
                    deg_out.at[c, pl.ds(s * RPT, RPT)])

  return pl.kernel(
      body,
      out_type=jax.ShapeDtypeStruct((NC, NP), _f32),
      mesh=_mesh,
      scratch_types=[
          pltpu.VMEM((CHUNK,), _f32),
          pltpu.VMEM((cpt, CHUNK), jnp.int32),
          pltpu.VMEM_SHARED((NP,), _f32),
      ],
  )


_deg_call = _make_deg()


def _make_agg(width):
  cpt = NCHUNK // NS
  nidx = 32
  ngrp = cpt // nidx

  def body(hs_hbm, src_hbm, dst_hbm, out_hbm,
           srcs_v, dsts_v, rows0, rows1, rows2, rows3,
           semg0, semg1, semg2, semg3, sems0, sems1, sems2, sems3,
           tab_sh, acc_sh):
    c = lax.axis_index("c")
    s = lax.axis_index("s")
    r0 = s * RPT
    pltpu.sync_copy(hs_hbm.at[c, pl.ds(r0, RPT)], tab_sh.at[pl.ds(r0, RPT)])
    pltpu.sync_copy(hs_hbm.at[c, pl.ds(r0, RPT)], acc_sh.at[pl.ds(r0, RPT)])
    plsc.subcore_barrier()
    base = s * cpt

    def start_g(buf, idx_row, semg):
      return pltpu.async_copy(tab_sh.at[idx_row], buf, semg)

    def wait_g(buf, semg):
      pltpu.make_async_copy(tab_sh.at[pl.ds(0, CHUNK)], buf, semg).wait()

    def start_s(buf, idx_row, sems):
      return pltpu.async_copy(buf, acc_sh.at[idx_row], sems, add=True)

    def wait_s(buf, sems):
      pltpu.make_async_copy(buf, acc_sh.at[pl.ds(0, CHUNK)], sems).wait()

    bufs = ((rows0, semg0, sems0), (rows1, semg1, sems1),
            (rows2, semg2, sems2), (rows3, semg3, sems3))
    nb = len(bufs)

    def group(g, carry):
      gbase = base + g * nidx
      pltpu.sync_copy(src_hbm.at[pl.ds(gbase, nidx)], srcs_v)
      pltpu.sync_copy(dst_hbm.at[pl.ds(gbase, nidx)], dsts_v)
      for b, (buf, sg, _) in enumerate(bufs):
        start_g(buf, srcs_v.at[b], sg)

      def quad(k, c2):
        j = nb * k + nb
        for b, (buf, sg, ss) in enumerate(bufs):
          wait_g(buf, sg)
          start_s(buf, dsts_v.at[j - nb + b], ss)
        for b, (buf, sg, ss) in enumerate(bufs):
          wait_s(buf, ss)
          start_g(buf, srcs_v.at[j + b], sg)
        return c2

      lax.fori_loop(0, nidx // nb - 1, quad, 0)
      for b, (buf, sg, ss) in enumerate(bufs):
        wait_g(buf, sg)
        start_s(buf, dsts_v.at[nidx - nb + b], ss)
      for b, (buf, sg, ss) in enumerate(bufs):
        wait_s(buf, ss)
      return carry

    lax.fori_loop(0, ngrp, group, 0)
    plsc.subcore_barrier()
    pltpu.sync_copy(acc_sh.at[pl.ds(r0, RPT)], out_hbm.at[c, pl.ds(r0, RPT)])

  return pl.kernel(
      body,
      out_type=jax.ShapeDtypeStruct((NC, NP, width), _f32),
      mesh=_mesh,
      compiler_params=_sc_params,
      scratch_types=[
          pltpu.VMEM((nidx, CHUNK), jnp.int32),
          pltpu.VMEM((nidx, CHUNK), jnp.int32),
          pltpu.VMEM((CHUNK, width), _f32),
          pltpu.VMEM((CHUNK, width), _f32),
          pltpu.VMEM((CHUNK, width), _f32),
          pltpu.VMEM((CHUNK, width), _f32),
          pltpu.SemaphoreType.DMA,
          pltpu.SemaphoreType.DMA,
          pltpu.SemaphoreType.DMA,
          pltpu.SemaphoreType.DMA,
          pltpu.SemaphoreType.DMA,
          pltpu.SemaphoreType.DMA,
          pltpu.SemaphoreType.DMA,
          pltpu.SemaphoreType.DMA,
          pltpu.VMEM_SHARED((NP, width), _f32),
          pltpu.VMEM_SHARED((NP, width), _f32),
      ],
  )


_agg64 = _make_agg(H1 // NC)
_agg32 = _make_agg(H2 // NC)


def _k2_body(deg_ref, x_ref, w1_ref, hs_ref, dinv_ref):
  d = deg_ref[...]
  deg = d[0] + d[1] + 1.0
  dinv = lax.rsqrt(deg)[:, None]
  h = jnp.dot(x_ref[...], w1_ref[...], preferred_element_type=_f32)
  hs = h * dinv
  hs_ref[0] = hs[:, : H1 // 2]
  hs_ref[1] = hs[:, H1 // 2:]
  dinv_ref[...] = dinv


def _k2_call(degp, xp, W1):
  return pl.pallas_call(
      _k2_body,
      grid=(GRID,),
      in_specs=[
          pl.BlockSpec((NC, RB), lambda i: (0, i)),
          pl.BlockSpec((RB, D_IN), lambda i: (i, 0)),
          pl.BlockSpec((D_IN, H1), lambda i: (0, 0)),
      ],
      out_specs=[
          pl.BlockSpec((NC, RB, H1 // 2), lambda i: (0, i, 0)),
          pl.BlockSpec((RB, 1), lambda i: (i, 0)),
      ],
      out_shape=[
          jax.ShapeDtypeStruct((NC, NP, H1 // 2), _f32),
          jax.ShapeDtypeStruct((NP, 1), _f32),
      ],
  )(degp, xp, W1)


def _k4_body(agg_ref, dinv_ref, b1_ref, w2_ref, out_ref):
  a = agg_ref[...]
  agg = jnp.concatenate([a[0], a[1]], axis=1)
  dinv = dinv_ref[...]
  h1 = jnp.maximum(agg * dinv + b1_ref[...], 0.0)
  gs = jnp.dot(h1, w2_ref[...], preferred_element_type=_f32) * dinv
  out_ref[0] = gs[:, : H2 // 2]
  out_ref[1] = gs[:, H2 // 2:]


def _k4_call(agg1, dinv, b1, W2):
  return pl.pallas_call(
      _k4_body,
      grid=(GRID,),
      in_specs=[
          pl.BlockSpec((NC, RB, H1 // 2), lambda i: (0, i, 0)),
          pl.BlockSpec((RB, 1), lambda i: (i, 0)),
          pl.BlockSpec((1, H1), lambda i: (0, 0)),
          pl.BlockSpec((H1, H2), lambda i: (0, 0)),
      ],
      out_specs=pl.BlockSpec((NC, RB, H2 // 2), lambda i: (0, i, 0)),
      out_shape=jax.ShapeDtypeStruct((NC, NP, H2 // 2), _f32),
  )(agg1, dinv, b1, W2)


def _k6_body(agg_ref, dinv_ref, b2_ref, out_ref):
  a = agg_ref[...]
  agg = jnp.concatenate([a[0], a[1]], axis=1)
  h2 = agg * dinv_ref[...] + b2_ref[...]
  out_ref[...] = jnp.maximum(h2, 0.0)


def _k6_call(agg2, dinv, b2):
  return pl.pallas_call(
      _k6_body,
      grid=(GRID,),
      in_specs=[
          pl.BlockSpec((NC, RB, H2 // 2), lambda i: (0, i, 0)),
          pl.BlockSpec((RB, 1), lambda i: (i, 0)),
          pl.BlockSpec((1, H2), lambda i: (0, 0)),
      ],
      out_specs=pl.BlockSpec((RB, H2), lambda i: (i, 0)),
      out_shape=jax.ShapeDtypeStruct((NP, H2), _f32),
  )(agg2, dinv, b2)


def _head_body(z_ref, wl_ref, bl_ref, out_ref):
  logits = jnp.dot(z_ref[...], wl_ref[...], preferred_element_type=_f32)
  logits = logits + bl_ref[...]
  m = jnp.max(logits, axis=1, keepdims=True)
  lse = jnp.log(jnp.sum(jnp.exp(logits - m), axis=1, keepdims=True)) + m
  out_ref[...] = logits - lse


def _head_call(z, Wl, bl):
  return pl.pallas_call(
      _head_body,
      out_shape=jax.ShapeDtypeStruct((NPG, OUT_DIM), _f32),
  )(z, Wl, bl)


def kernel(x, edge_index, W1, b1, W2, b2, Wl, bl):
  xp = jnp.concatenate([x, jnp.zeros((NP - N, D_IN), _f32)], axis=0)
  pad = jnp.full((EP - E,), DUMMY, jnp.int32)
  src2 = jnp.concatenate([edge_index[0], pad]).reshape(NCHUNK, CHUNK)
  dst2 = jnp.concatenate([edge_index[1], pad]).reshape(NCHUNK, CHUNK)
  ones_rows = jnp.ones((CHUNK,), _f32)
  zeros_deg = jnp.zeros((NP,), _f32)

  degp = _deg_call(dst2, ones_rows, zeros_deg)
  hs, dinv = _k2_call(degp, xp, W1)
  agg1 = _agg64(hs, src2, dst2)
  gs = _k4_call(agg1, dinv, b1.reshape(1, H1), W2)
  agg2 = _agg32(gs, src2, dst2)
  h2 = _k6_call(agg2, dinv, b2.reshape(1, H2))
  z = h2[:N].reshape(NPG, H2 * NPG)
  return _head_call(z, Wl, bl.reshape(1, OUT_DIM))
